# baseline (speedup 1.0000x reference)
import functools

import jax
import jax.numpy as jnp
from jax import lax
from jax.experimental import pallas as pl
from jax.experimental.pallas import tpu as pltpu
from jax.experimental.pallas import tpu_sc as plsc

NC = 2
NS = 16
NW = NC * NS

CHUNK = 1600
SPLIT = 4


@functools.lru_cache(maxsize=None)
def _build(N: int, V: int, D: int):
    n_per_w = N // NW
    n_chunks = n_per_w // CHUNK
    mesh = plsc.VectorSubcoreMesh(core_axis_name="c", subcore_axis_name="s")

    @functools.partial(
        pl.kernel,
        mesh=mesh,
        compiler_params=pltpu.CompilerParams(use_tc_tiling_on_sc=False),
        out_type=jax.ShapeDtypeStruct((N, D), jnp.float32),
        scratch_types=[
            pltpu.VMEM((n_per_w,), jnp.int32),
            pltpu.VMEM((CHUNK, D), jnp.float32),
            pltpu.VMEM((CHUNK, D), jnp.float32),
            pltpu.SemaphoreType.DMA,
            pltpu.SemaphoreType.DMA,
            pltpu.SemaphoreType.DMA,
            pltpu.SemaphoreType.DMA,
        ],
    )
    def gather_kernel(table_hbm, idx_hbm, out_hbm,
                      idx_v, rows0, rows1, g0, g1, o0, o1):
        wid = lax.axis_index("s") * NC + lax.axis_index("c")
        w_base = wid * n_per_w
        rows = (rows0, rows1)
        gsem = (g0, g1)
        osem = (o0, o1)

        pltpu.sync_copy(idx_hbm.at[pl.ds(w_base, n_per_w)], idx_v)

        def gather(i, b):
            return pltpu.async_copy(
                table_hbm.at[idx_v.at[pl.ds(i * CHUNK, CHUNK)]],
                rows[b], gsem[b])

        def writeback(i, b):
            return pltpu.async_copy(
                rows[b], out_hbm.at[pl.ds(w_base + i * CHUNK, CHUNK)],
                osem[b])

        pending_g = gather(0, 0)
        pending_o = [None, None]
        for i in range(n_chunks):
            b = i % 2
            pending_g.wait()
            if i + 1 < n_chunks:
                if pending_o[1 - b] is not None:
                    pending_o[1 - b].wait()
                pending_g = gather(i + 1, 1 - b)
            pending_o[b] = writeback(i, b)
        for p in pending_o:
            if p is not None:
                p.wait()

    return gather_kernel


def kernel(x, table):
    Bq, Lq = x.shape
    V, D = table.shape
    N = Bq * Lq
    idx = x.reshape(N).astype(jnp.int32)
    npiece = N // SPLIT
    bpiece = Bq // SPLIT
    gk = _build(npiece, V, D)
    pieces = [
        gk(table, lax.slice(idx, (i * npiece,), ((i + 1) * npiece,)))
        .reshape(bpiece, Lq, D)
        for i in range(SPLIT)
    ]
    return jnp.concatenate(pieces, axis=0)
